# Initial kernel scaffold; baseline (speedup 1.0000x reference)
import jax
import jax.numpy as jnp
from jax.experimental import pallas as pl


def kernel(x_nchw, w1, b1, w2, b2):
    raise NotImplementedError("write your pallas kernel here")


# speedup vs baseline: 1.2403x; 1.2403x over previous
import functools

import jax
import jax.numpy as jnp
from jax.experimental import pallas as pl
from jax.experimental.pallas import tpu as pltpu


def _se_kernel(x_ref, w1_ref, b1_ref, w2_ref, b2_ref, o_ref, *, inv_hw):
    x = x_ref[0]
    pooled = jnp.sum(x, axis=-1, keepdims=True,
                     dtype=jnp.float32) * inv_hw
    h = jnp.dot(w1_ref[...], pooled,
                preferred_element_type=jnp.float32) + b1_ref[...]
    h = jnp.maximum(h, 0.0)
    logits = jnp.dot(w2_ref[...], h,
                     preferred_element_type=jnp.float32) + b2_ref[...]
    att = jax.nn.sigmoid(logits).astype(o_ref.dtype)
    o_ref[0] = x * att


def kernel(x_nchw, w1, b1, w2, b2):
    N, C, H, W = x_nchw.shape
    Cr = w1.shape[0]
    HW = H * W

    x = x_nchw.reshape(N, C, HW)
    itemsize = x.dtype.itemsize
    tile_bytes = C * HW * itemsize

    vmem_limit = int(min(max(6 * tile_bytes + (2 << 20), 8 << 20), 64 << 20))

    out = pl.pallas_call(
        functools.partial(_se_kernel, inv_hw=1.0 / HW),
        out_shape=jax.ShapeDtypeStruct((N, C, HW), x.dtype),
        grid_spec=pltpu.PrefetchScalarGridSpec(
            num_scalar_prefetch=0,
            grid=(N,),
            in_specs=[
                pl.BlockSpec((1, C, HW), lambda n: (n, 0, 0)),
                pl.BlockSpec((Cr, C), lambda n: (0, 0)),
                pl.BlockSpec((Cr, 1), lambda n: (0, 0)),
                pl.BlockSpec((C, Cr), lambda n: (0, 0)),
                pl.BlockSpec((C, 1), lambda n: (0, 0)),
            ],
            out_specs=pl.BlockSpec((1, C, HW), lambda n: (n, 0, 0)),
        ),
        compiler_params=pltpu.CompilerParams(
            dimension_semantics=("parallel",),
            vmem_limit_bytes=vmem_limit),
        cost_estimate=pl.CostEstimate(
            flops=2 * N * C * HW + 4 * N * C * Cr,
            transcendentals=N * C,
            bytes_accessed=2 * N * C * HW * itemsize
            + (2 * C * Cr + C + Cr) * 4),
    )(x, w1, b1.reshape(Cr, 1), w2, b2.reshape(C, 1))

    return out.reshape(N, C, H, W)
